# Initial kernel scaffold; baseline (speedup 1.0000x reference)
import functools

import jax
import jax.numpy as jnp
from jax import lax
from jax.experimental import pallas as pl
from jax.experimental.pallas import tpu as pltpu
from jax.experimental.pallas import tpu_sc as plsc

N = 10000
E = 320000
D = 128

NC = 2
NS = 16
NW = NC * NS
C = 128

N_PAD = 10240
ROWS_PER_SUB = N_PAD // NS
E_PAD = ((E + NW * C - 1) // (NW * C)) * (NW * C)
R = E_PAD // C
K = R // NW

_mesh = plsc.VectorSubcoreMesh(core_axis_name="c", subcore_axis_name="s")


@functools.partial(
    pl.kernel,
    out_type=jax.ShapeDtypeStruct((NC, N_PAD), jnp.float32),
    mesh=_mesh,
    scratch_types=[
        pltpu.VMEM((C,), jnp.int32),
        pltpu.VMEM((C,), jnp.float32),
        pltpu.VMEM((ROWS_PER_SUB,), jnp.float32),
        pltpu.VMEM_SHARED((N_PAD,), jnp.float32),
    ],
)
def _sc_deg(dst2d, w2d, deg_out, dst_v, w_v, zline_v, deg_sh):
    cid = lax.axis_index("c")
    sid = lax.axis_index("s")
    wid = sid * NC + cid
    zofs = sid * ROWS_PER_SUB
    for j in range(ROWS_PER_SUB // 16):
        zline_v[pl.ds(j * 16, 16)] = jnp.zeros((16,), jnp.float32)
    pltpu.sync_copy(zline_v, deg_sh.at[pl.ds(zofs, ROWS_PER_SUB)])
    plsc.subcore_barrier()

    def chunk(k, carry):
        row = wid * K + k
        pltpu.sync_copy(dst2d.at[row], dst_v)
        pltpu.sync_copy(w2d.at[row], w_v)
        pltpu.sync_copy(w_v, deg_sh.at[dst_v], add=True)
        return carry

    lax.fori_loop(0, K, chunk, 0)
    plsc.subcore_barrier()
    pltpu.sync_copy(deg_sh.at[pl.ds(zofs, ROWS_PER_SUB)],
                    deg_out.at[cid, pl.ds(zofs, ROWS_PER_SUB)])


@functools.partial(
    pl.kernel,
    out_type=jax.ShapeDtypeStruct((NC, N_PAD, D), jnp.float32),
    mesh=_mesh,
    scratch_types=[
        pltpu.VMEM((C,), jnp.int32),
        pltpu.VMEM((C,), jnp.int32),
        pltpu.VMEM((C,), jnp.float32),
        pltpu.VMEM((C, D), jnp.float32),
        pltpu.SemaphoreType.DMA,
        pltpu.VMEM_SHARED((N_PAD, D), jnp.float32),
    ],
)
def _sc_msg(g_hbm, src2d, dst2d, w2d, z_hbm, acc_out,
            src_v, dst_v, w_v, rows_v, sem, acc_sh):
    cid = lax.axis_index("c")
    sid = lax.axis_index("s")
    wid = sid * NC + cid
    zofs = sid * ROWS_PER_SUB

    @pl.when(cid == 0)
    def _():
        pltpu.sync_copy(g_hbm.at[pl.ds(zofs, ROWS_PER_SUB)],
                        acc_sh.at[pl.ds(zofs, ROWS_PER_SUB)])

    @pl.when(cid != 0)
    def _():
        pltpu.sync_copy(z_hbm.at[pl.ds(zofs, ROWS_PER_SUB)],
                        acc_sh.at[pl.ds(zofs, ROWS_PER_SUB)])

    plsc.subcore_barrier()

    def chunk(k, carry):
        row = wid * K + k
        pltpu.sync_copy(src2d.at[row], src_v)
        pltpu.sync_copy(dst2d.at[row], dst_v)
        pltpu.sync_copy(w2d.at[row], w_v)
        pltpu.async_copy(g_hbm.at[src_v], rows_v, sem).wait()

        def mul_e(e, c2):
            splat = plsc.load_gather(
                w_v, [jnp.full((16,), 0, jnp.int32) + e])
            for j in range(D // 16):
                sl = pl.ds(j * 16, 16)
                rows_v[e, sl] = rows_v[e, sl] * splat
            return c2

        lax.fori_loop(0, C, mul_e, 0)
        pltpu.sync_copy(rows_v, acc_sh.at[dst_v], add=True)
        return carry

    lax.fori_loop(0, K, chunk, 0)
    plsc.subcore_barrier()
    pltpu.sync_copy(acc_sh.at[pl.ds(zofs, ROWS_PER_SUB)],
                    acc_out.at[cid, pl.ds(zofs, ROWS_PER_SUB)])


def _tc_dis_body(deg_ref, out_ref):
    out_ref[...] = lax.rsqrt(1.0 + deg_ref[0:1, :] + deg_ref[1:2, :])


def _tc_dis(deg2):
    return pl.pallas_call(
        _tc_dis_body,
        out_shape=jax.ShapeDtypeStruct((1, N_PAD), jnp.float32),
    )(deg2)


_MM_BLK = 640


def _tc_mm_body(x_ref, w_ref, dis_ref, g_ref):
    mm = jnp.dot(x_ref[...], w_ref[...], preferred_element_type=jnp.float32)
    g_ref[...] = mm * dis_ref[...]


def _tc_mm(x_p, W, dis_col):
    grid = (N_PAD // _MM_BLK,)
    return pl.pallas_call(
        _tc_mm_body,
        grid=grid,
        in_specs=[
            pl.BlockSpec((_MM_BLK, D), lambda i: (i, 0)),
            pl.BlockSpec((D, D), lambda i: (0, 0)),
            pl.BlockSpec((_MM_BLK, 1), lambda i: (i, 0)),
        ],
        out_specs=pl.BlockSpec((_MM_BLK, D), lambda i: (i, 0)),
        out_shape=jax.ShapeDtypeStruct((N_PAD, D), jnp.float32),
    )(x_p, W, dis_col)


def _tc_fin_body(a0_ref, a1_ref, dis_ref, b_ref, pa_ref, out_ref):
    o = dis_ref[...] * (a0_ref[...] + a1_ref[...]) + b_ref[...]
    out_ref[...] = jnp.where(o >= 0.0, o, pa_ref[...] * o)


def _tc_fin(acc0, acc1, dis_col, b2, pa2):
    grid = (N_PAD // _MM_BLK,)
    return pl.pallas_call(
        _tc_fin_body,
        grid=grid,
        in_specs=[
            pl.BlockSpec((_MM_BLK, D), lambda i: (i, 0)),
            pl.BlockSpec((_MM_BLK, D), lambda i: (i, 0)),
            pl.BlockSpec((_MM_BLK, 1), lambda i: (i, 0)),
            pl.BlockSpec((1, D), lambda i: (0, 0)),
            pl.BlockSpec((1, D), lambda i: (0, 0)),
        ],
        out_specs=pl.BlockSpec((_MM_BLK, D), lambda i: (i, 0)),
        out_shape=jax.ShapeDtypeStruct((N_PAD, D), jnp.float32),
    )(acc0, acc1, dis_col, b2, pa2)


@jax.jit
def kernel(x, edge_index, edge_weight, W, b, prelu_a):
    src = edge_index[0]
    dst = edge_index[1]
    pad_e = E_PAD - E
    src2d = jnp.concatenate(
        [src, jnp.zeros((pad_e,), jnp.int32)]).reshape(R, C)
    dst2d = jnp.concatenate(
        [dst, jnp.zeros((pad_e,), jnp.int32)]).reshape(R, C)
    w2d = jnp.concatenate(
        [edge_weight, jnp.zeros((pad_e,), jnp.float32)]).reshape(R, C)
    x_p = jnp.concatenate(
        [x, jnp.zeros((N_PAD - N, D), jnp.float32)], axis=0)
    z_nd = jnp.zeros((N_PAD, D), jnp.float32)

    deg2 = _sc_deg(dst2d, w2d)
    dis_col = _tc_dis(deg2).reshape(N_PAD, 1)
    g = _tc_mm(x_p, W, dis_col)
    acc2 = _sc_msg(g, src2d, dst2d, w2d, z_nd)
    out = _tc_fin(acc2[0], acc2[1], dis_col,
                  b.reshape(1, D), prelu_a.reshape(1, D))
    return out[:N]


# speedup vs baseline: 11.7822x; 11.7822x over previous
import functools

import jax
import jax.numpy as jnp
from jax import lax
from jax.experimental import pallas as pl
from jax.experimental.pallas import tpu as pltpu
from jax.experimental.pallas import tpu_sc as plsc

N = 10000
E = 320000
D = 128

NC = 2
NS = 16
NW = NC * NS
C = 128

N_PAD = 10240
ROWS_PER_SUB = N_PAD // NS
E_PAD = ((E + NW * C - 1) // (NW * C)) * (NW * C)
R = E_PAD // C
K = R // NW

_mesh = plsc.VectorSubcoreMesh(core_axis_name="c", subcore_axis_name="s")
_sc_params = pltpu.CompilerParams(needs_layout_passes=False)


@functools.partial(
    pl.kernel,
    out_type=jax.ShapeDtypeStruct((NC, N_PAD), jnp.float32),
    mesh=_mesh,
    scratch_types=[
        pltpu.VMEM((C,), jnp.int32),
        pltpu.VMEM((C,), jnp.float32),
        pltpu.VMEM((ROWS_PER_SUB,), jnp.float32),
        pltpu.VMEM_SHARED((N_PAD,), jnp.float32),
    ],
    compiler_params=_sc_params,
)
def _sc_deg(dst2d, w2d, deg_out, dst_v, w_v, zline_v, deg_sh):
    cid = lax.axis_index("c")
    sid = lax.axis_index("s")
    wid = sid * NC + cid
    zofs = sid * ROWS_PER_SUB
    for j in range(ROWS_PER_SUB // 16):
        zline_v[pl.ds(j * 16, 16)] = jnp.zeros((16,), jnp.float32)
    pltpu.sync_copy(zline_v, deg_sh.at[pl.ds(zofs, ROWS_PER_SUB)])
    plsc.subcore_barrier()

    def chunk(k, carry):
        row = wid * K + k
        pltpu.sync_copy(dst2d.at[row], dst_v)
        pltpu.sync_copy(w2d.at[row], w_v)
        pltpu.sync_copy(w_v, deg_sh.at[dst_v], add=True)
        return carry

    lax.fori_loop(0, K, chunk, 0)
    plsc.subcore_barrier()
    pltpu.sync_copy(deg_sh.at[pl.ds(zofs, ROWS_PER_SUB)],
                    deg_out.at[cid, pl.ds(zofs, ROWS_PER_SUB)])


@functools.partial(
    pl.kernel,
    out_type=jax.ShapeDtypeStruct((NC, N_PAD, D), jnp.float32),
    mesh=_mesh,
    scratch_types=[
        pltpu.VMEM((C,), jnp.int32),
        pltpu.VMEM((C,), jnp.int32),
        pltpu.VMEM((C,), jnp.float32),
        pltpu.VMEM((C, D), jnp.float32),
        pltpu.SemaphoreType.DMA,
        pltpu.VMEM_SHARED((N_PAD, D), jnp.float32),
    ],
    compiler_params=_sc_params,
)
def _sc_msg(g_hbm, src2d, dst2d, w2d, z_hbm, acc_out,
            src_v, dst_v, w_v, rows_v, sem, acc_sh):
    cid = lax.axis_index("c")
    sid = lax.axis_index("s")
    wid = sid * NC + cid
    zofs = sid * ROWS_PER_SUB

    @pl.when(cid == 0)
    def _():
        pltpu.sync_copy(g_hbm.at[pl.ds(zofs, ROWS_PER_SUB)],
                        acc_sh.at[pl.ds(zofs, ROWS_PER_SUB)])

    @pl.when(cid != 0)
    def _():
        pltpu.sync_copy(z_hbm.at[pl.ds(zofs, ROWS_PER_SUB)],
                        acc_sh.at[pl.ds(zofs, ROWS_PER_SUB)])

    plsc.subcore_barrier()

    def chunk(k, carry):
        row = wid * K + k
        pltpu.sync_copy(src2d.at[row], src_v)
        pltpu.sync_copy(dst2d.at[row], dst_v)
        pltpu.sync_copy(w2d.at[row], w_v)
        pltpu.async_copy(g_hbm.at[src_v], rows_v, sem).wait()

        def mul_e(e, c2):
            splat = plsc.load_gather(
                w_v, [jnp.full((16,), 0, jnp.int32) + e])
            for j in range(D // 16):
                sl = pl.ds(j * 16, 16)
                rows_v[e, sl] = rows_v[e, sl] * splat
            return c2

        lax.fori_loop(0, C, mul_e, 0)
        pltpu.sync_copy(rows_v, acc_sh.at[dst_v], add=True)
        return carry

    lax.fori_loop(0, K, chunk, 0)
    plsc.subcore_barrier()
    pltpu.sync_copy(acc_sh.at[pl.ds(zofs, ROWS_PER_SUB)],
                    acc_out.at[cid, pl.ds(zofs, ROWS_PER_SUB)])


def _tc_dis_body(deg_ref, out_ref):
    out_ref[...] = lax.rsqrt(1.0 + deg_ref[0:1, :] + deg_ref[1:2, :])


def _tc_dis(deg2):
    return pl.pallas_call(
        _tc_dis_body,
        out_shape=jax.ShapeDtypeStruct((1, N_PAD), jnp.float32),
    )(deg2)


_MM_BLK = 640


def _tc_mm_body(x_ref, w_ref, dis_ref, g_ref):
    mm = jnp.dot(x_ref[...], w_ref[...], preferred_element_type=jnp.float32)
    g_ref[...] = mm * dis_ref[...]


def _tc_mm(x_p, W, dis_col):
    grid = (N_PAD // _MM_BLK,)
    return pl.pallas_call(
        _tc_mm_body,
        grid=grid,
        in_specs=[
            pl.BlockSpec((_MM_BLK, D), lambda i: (i, 0)),
            pl.BlockSpec((D, D), lambda i: (0, 0)),
            pl.BlockSpec((_MM_BLK, 1), lambda i: (i, 0)),
        ],
        out_specs=pl.BlockSpec((_MM_BLK, D), lambda i: (i, 0)),
        out_shape=jax.ShapeDtypeStruct((N_PAD, D), jnp.float32),
    )(x_p, W, dis_col)


def _tc_fin_body(a0_ref, a1_ref, dis_ref, b_ref, pa_ref, out_ref):
    o = dis_ref[...] * (a0_ref[...] + a1_ref[...]) + b_ref[...]
    out_ref[...] = jnp.where(o >= 0.0, o, pa_ref[...] * o)


def _tc_fin(acc0, acc1, dis_col, b2, pa2):
    grid = (N_PAD // _MM_BLK,)
    return pl.pallas_call(
        _tc_fin_body,
        grid=grid,
        in_specs=[
            pl.BlockSpec((_MM_BLK, D), lambda i: (i, 0)),
            pl.BlockSpec((_MM_BLK, D), lambda i: (i, 0)),
            pl.BlockSpec((_MM_BLK, 1), lambda i: (i, 0)),
            pl.BlockSpec((1, D), lambda i: (0, 0)),
            pl.BlockSpec((1, D), lambda i: (0, 0)),
        ],
        out_specs=pl.BlockSpec((_MM_BLK, D), lambda i: (i, 0)),
        out_shape=jax.ShapeDtypeStruct((N_PAD, D), jnp.float32),
    )(acc0, acc1, dis_col, b2, pa2)


@jax.jit
def kernel(x, edge_index, edge_weight, W, b, prelu_a):
    src = edge_index[0]
    dst = edge_index[1]
    pad_e = E_PAD - E
    src2d = jnp.concatenate(
        [src, jnp.zeros((pad_e,), jnp.int32)]).reshape(R, C)
    dst2d = jnp.concatenate(
        [dst, jnp.zeros((pad_e,), jnp.int32)]).reshape(R, C)
    w2d = jnp.concatenate(
        [edge_weight, jnp.zeros((pad_e,), jnp.float32)]).reshape(R, C)
    x_p = jnp.concatenate(
        [x, jnp.zeros((N_PAD - N, D), jnp.float32)], axis=0)
    z_nd = jnp.zeros((N_PAD, D), jnp.float32)

    deg2 = _sc_deg(dst2d, w2d)
    dis_col = _tc_dis(deg2).reshape(N_PAD, 1)
    g = _tc_mm(x_p, W, dis_col)
    acc2 = _sc_msg(g, src2d, dst2d, w2d, z_nd)
    out = _tc_fin(acc2[0], acc2[1], dis_col,
                  b.reshape(1, D), prelu_a.reshape(1, D))
    return out[:N]


# speedup vs baseline: 23.5280x; 1.9969x over previous
import functools

import jax
import jax.numpy as jnp
from jax import lax
from jax.experimental import pallas as pl
from jax.experimental.pallas import tpu as pltpu
from jax.experimental.pallas import tpu_sc as plsc

N = 10000
E = 320000
D = 128

NC = 2
NS = 16
NW = NC * NS
C = 128
NB = 4

N_PAD = 10240
ROWS_PER_SUB = N_PAD // NS
E_PAD = ((E + NW * C * NB - 1) // (NW * C * NB)) * (NW * C * NB)
R = E_PAD // C
K = R // NW
KG = K // NB

CM = 112
NBM = 3
KM = 90
RM = NW * KM
EM_PAD = RM * CM

_mesh = plsc.VectorSubcoreMesh(core_axis_name="c", subcore_axis_name="s")
_sc_params = pltpu.CompilerParams(needs_layout_passes=False)

@functools.partial(
    pl.kernel,
    out_type=jax.ShapeDtypeStruct((NC, N_PAD), jnp.float32),
    mesh=_mesh,
    scratch_types=[
        pltpu.VMEM((K, C), jnp.int32),
        pltpu.VMEM((K, C), jnp.float32),
        pltpu.VMEM((ROWS_PER_SUB,), jnp.float32),
        pltpu.VMEM_SHARED((N_PAD,), jnp.float32),
        pltpu.SemaphoreType.DMA,
        pltpu.SemaphoreType.DMA,
        pltpu.SemaphoreType.DMA,
        pltpu.SemaphoreType.DMA,
    ],
    compiler_params=_sc_params,
)
def _sc_deg(dst2d, w2d, deg_out, dst_all, w_all, zline_v, deg_sh,
            s0, s1, s2, s3):
    sems = (s0, s1, s2, s3)
    cid = lax.axis_index("c")
    sid = lax.axis_index("s")
    wid = sid * NC + cid
    zofs = sid * ROWS_PER_SUB
    def zrow(i, carry):
        zline_v[pl.ds(i * 16, 16)] = jnp.zeros((16,), jnp.float32)
        return carry
    lax.fori_loop(0, ROWS_PER_SUB // 16, zrow, 0)
    pltpu.sync_copy(zline_v, deg_sh.at[pl.ds(zofs, ROWS_PER_SUB)])
    pltpu.sync_copy(dst2d.at[pl.ds(wid * K, K)], dst_all)
    pltpu.sync_copy(w2d.at[pl.ds(wid * K, K)], w_all)
    plsc.subcore_barrier()

    def chunk(k, carry):
        pltpu.sync_copy(w_all.at[k], deg_sh.at[dst_all.at[k]], add=True)
        return carry

    lax.fori_loop(0, K, chunk, 0)
    plsc.subcore_barrier()
    pltpu.sync_copy(deg_sh.at[pl.ds(zofs, ROWS_PER_SUB)],
                    deg_out.at[cid, pl.ds(zofs, ROWS_PER_SUB)])


@functools.partial(
    pl.kernel,
    out_type=jax.ShapeDtypeStruct((NC, N_PAD, D), jnp.float32),
    mesh=_mesh,
    scratch_types=[
        [pltpu.VMEM((CM, D), jnp.float32) for _ in range(NBM)],
        [pltpu.VMEM((CM,), jnp.int32) for _ in range(NBM)],
        [pltpu.VMEM((CM,), jnp.int32) for _ in range(NBM)],
        [pltpu.VMEM((CM,), jnp.float32) for _ in range(NBM)],
        [pltpu.SemaphoreType.DMA for _ in range(NBM)],
        [pltpu.SemaphoreType.DMA for _ in range(NBM)],
        [pltpu.SemaphoreType.DMA for _ in range(NBM)],
        [pltpu.SemaphoreType.DMA for _ in range(NBM)],
        [pltpu.SemaphoreType.DMA for _ in range(NBM)],
        pltpu.VMEM_SHARED((N_PAD, D), jnp.float32),
    ],
    compiler_params=_sc_params,
)
def _sc_msg(g_hbm, src2d, dst2d, w2d, acc_out,
            rows, srcb, dstb, wb, gsem, ssem, srcsem, dstsem, wsem, acc_sh):
    cid = lax.axis_index("c")
    sid = lax.axis_index("s")
    wid = sid * NC + cid
    zofs = sid * ROWS_PER_SUB
    base = wid * KM

    def zrow(i, carry):
        for j in range(D // 16):
            rows[0][i, pl.ds(j * 16, 16)] = jnp.zeros((16,), jnp.float32)
        return carry
    lax.fori_loop(0, 64, zrow, 0)
    for i in range(ROWS_PER_SUB // 64):
        pltpu.sync_copy(rows[0].at[pl.ds(0, 64)],
                        acc_sh.at[pl.ds(zofs + i * 64, 64)])

    for b in range(NBM):
        pltpu.sync_copy(src2d.at[base + b], srcb[b])
        pltpu.sync_copy(w2d.at[base + b], wb[b])
    for b in range(NBM - 1):
        pltpu.sync_copy(dst2d.at[base + b], dstb[b])
    plsc.subcore_barrier()
    for b in range(NBM - 1):
        pltpu.async_copy(g_hbm.at[srcb[b]], rows[b], gsem[b])

    def group(g, carry):
        for b in range(NBM):
            k = g * NBM + b
            b2 = (b + 2) % NBM
            pltpu.make_async_copy(g_hbm.at[srcb[b]], rows[b], gsem[b]).wait()

            def mul_e(e, c2):
                splat = plsc.load_gather(
                    wb[b], [jnp.full((16,), 0, jnp.int32) + e])
                for j in range(D // 16):
                    sl = pl.ds(j * 16, 16)
                    rows[b][e, sl] = rows[b][e, sl] * splat
                return c2
            lax.fori_loop(0, CM, mul_e, 0)

            @pl.when(k >= 2)
            def _():
                pltpu.make_async_copy(
                    dst2d.at[base], dstb[b], dstsem[b]).wait()
            pltpu.async_copy(rows[b], acc_sh.at[dstb[b]], ssem[b], add=True)

            @pl.when(k >= 1)
            def _():
                pltpu.make_async_copy(
                    rows[b2], acc_sh.at[dstb[b2]], ssem[b2]).wait()

            @pl.when(k + 2 < KM)
            def _():
                @pl.when(k >= 1)
                def _():
                    pltpu.make_async_copy(
                        src2d.at[base], srcb[b2], srcsem[b2]).wait()
                    pltpu.make_async_copy(
                        w2d.at[base], wb[b2], wsem[b2]).wait()
                pltpu.async_copy(g_hbm.at[srcb[b2]], rows[b2], gsem[b2])
                pltpu.async_copy(
                    dst2d.at[base + k + 2], dstb[b2], dstsem[b2])

            @pl.when(k + 3 < KM)
            def _():
                pltpu.async_copy(src2d.at[base + k + 3], srcb[b], srcsem[b])
                pltpu.async_copy(w2d.at[base + k + 3], wb[b], wsem[b])
        return carry

    lax.fori_loop(0, KM // NBM, group, 0)
    bl = (KM - 1) % NBM
    pltpu.make_async_copy(rows[bl], acc_sh.at[dstb[bl]], ssem[bl]).wait()
    plsc.subcore_barrier()
    pltpu.sync_copy(acc_sh.at[pl.ds(zofs, ROWS_PER_SUB)],
                    acc_out.at[cid, pl.ds(zofs, ROWS_PER_SUB)])


def _tc_dis_body(deg_ref, out_ref):
    out_ref[...] = lax.rsqrt(1.0 + deg_ref[0:1, :] + deg_ref[1:2, :])


def _tc_dis(deg2):
    return pl.pallas_call(
        _tc_dis_body,
        out_shape=jax.ShapeDtypeStruct((1, N_PAD), jnp.float32),
    )(deg2)


_MM_BLK = 640


def _tc_mm_body(x_ref, w_ref, dis_ref, g_ref):
    mm = jnp.dot(x_ref[...], w_ref[...], preferred_element_type=jnp.float32)
    g_ref[...] = mm * dis_ref[...]


def _tc_mm(x_p, W, dis_col):
    grid = (N_PAD // _MM_BLK,)
    return pl.pallas_call(
        _tc_mm_body,
        grid=grid,
        in_specs=[
            pl.BlockSpec((_MM_BLK, D), lambda i: (i, 0)),
            pl.BlockSpec((D, D), lambda i: (0, 0)),
            pl.BlockSpec((_MM_BLK, 1), lambda i: (i, 0)),
        ],
        out_specs=pl.BlockSpec((_MM_BLK, D), lambda i: (i, 0)),
        out_shape=jax.ShapeDtypeStruct((N_PAD, D), jnp.float32),
    )(x_p, W, dis_col)


def _tc_fin_body(a0_ref, a1_ref, g_ref, dis_ref, b_ref, pa_ref, out_ref):
    o = dis_ref[...] * (a0_ref[...] + a1_ref[...] + g_ref[...]) + b_ref[...]
    out_ref[...] = jnp.where(o >= 0.0, o, pa_ref[...] * o)


def _tc_fin(acc0, acc1, g, dis_col, b2, pa2):
    grid = (N_PAD // _MM_BLK,)
    return pl.pallas_call(
        _tc_fin_body,
        grid=grid,
        in_specs=[
            pl.BlockSpec((_MM_BLK, D), lambda i: (i, 0)),
            pl.BlockSpec((_MM_BLK, D), lambda i: (i, 0)),
            pl.BlockSpec((_MM_BLK, D), lambda i: (i, 0)),
            pl.BlockSpec((_MM_BLK, 1), lambda i: (i, 0)),
            pl.BlockSpec((1, D), lambda i: (0, 0)),
            pl.BlockSpec((1, D), lambda i: (0, 0)),
        ],
        out_specs=pl.BlockSpec((_MM_BLK, D), lambda i: (i, 0)),
        out_shape=jax.ShapeDtypeStruct((N_PAD, D), jnp.float32),
    )(acc0, acc1, g, dis_col, b2, pa2)


@jax.jit
def kernel(x, edge_index, edge_weight, W, b, prelu_a):
    src = edge_index[0]
    dst = edge_index[1]
    pad_e = E_PAD - E
    dst2d = jnp.concatenate(
        [dst, jnp.zeros((pad_e,), jnp.int32)]).reshape(R, C)
    w2d = jnp.concatenate(
        [edge_weight, jnp.zeros((pad_e,), jnp.float32)]).reshape(R, C)
    pad_m = EM_PAD - E
    src2m = jnp.concatenate(
        [src, jnp.zeros((pad_m,), jnp.int32)]).reshape(RM, CM)
    dst2m = jnp.concatenate(
        [dst, jnp.zeros((pad_m,), jnp.int32)]).reshape(RM, CM)
    w2m = jnp.concatenate(
        [edge_weight, jnp.zeros((pad_m,), jnp.float32)]).reshape(RM, CM)
    x_p = jnp.concatenate(
        [x, jnp.zeros((N_PAD - N, D), jnp.float32)], axis=0)

    deg2 = _sc_deg(dst2d, w2d)
    dis_col = _tc_dis(deg2).reshape(N_PAD, 1)
    g = _tc_mm(x_p, W, dis_col)
    acc2 = _sc_msg(g, src2m, dst2m, w2m)
    out = _tc_fin(acc2[0], acc2[1], g, dis_col,
                  b.reshape(1, D), prelu_a.reshape(1, D))
    return out[:N]


# speedup vs baseline: 29.2412x; 1.2428x over previous
import functools

import jax
import jax.numpy as jnp
from jax import lax
from jax.experimental import pallas as pl
from jax.experimental.pallas import tpu as pltpu
from jax.experimental.pallas import tpu_sc as plsc

N = 10000
E = 320000
D = 128

NC = 2
NS = 16
NW = NC * NS

N_PAD = 10240
ROWS_PER_SUB = N_PAD // NS

CD = 125
KD = E // (NW * CD)
RD = E // CD

CM = 80
RM = E // CM
NBM = 3
KM0 = 152
KM1 = 98
KG0 = (KM0 - 2) // NBM
KG1 = (KM1 - 2) // NBM

_mesh = plsc.VectorSubcoreMesh(core_axis_name="c", subcore_axis_name="s")
_sc_params = pltpu.CompilerParams(needs_layout_passes=False)


@functools.partial(
    pl.kernel,
    out_type=jax.ShapeDtypeStruct((NC, N_PAD), jnp.float32),
    mesh=_mesh,
    scratch_types=[
        pltpu.VMEM((KD, CD), jnp.int32),
        pltpu.VMEM((KD, CD), jnp.float32),
        pltpu.VMEM((ROWS_PER_SUB,), jnp.float32),
        pltpu.VMEM_SHARED((N_PAD,), jnp.float32),
    ],
    compiler_params=_sc_params,
)
def _sc_deg(dst2d, w2d, deg_out, dst_all, w_all, zline_v, deg_sh):
    cid = lax.axis_index("c")
    sid = lax.axis_index("s")
    wid = sid * NC + cid
    zofs = sid * ROWS_PER_SUB

    def zrow(i, carry):
        zline_v[pl.ds(i * 16, 16)] = jnp.zeros((16,), jnp.float32)
        return carry
    lax.fori_loop(0, ROWS_PER_SUB // 16, zrow, 0)
    pltpu.sync_copy(zline_v, deg_sh.at[pl.ds(zofs, ROWS_PER_SUB)])
    pltpu.sync_copy(dst2d.at[pl.ds(wid * KD, KD)], dst_all)
    pltpu.sync_copy(w2d.at[pl.ds(wid * KD, KD)], w_all)
    plsc.subcore_barrier()

    def chunk(k, carry):
        pltpu.sync_copy(w_all.at[k], deg_sh.at[dst_all.at[k]], add=True)
        return carry

    lax.fori_loop(0, KD, chunk, 0)
    plsc.subcore_barrier()
    pltpu.sync_copy(deg_sh.at[pl.ds(zofs, ROWS_PER_SUB)],
                    deg_out.at[cid, pl.ds(zofs, ROWS_PER_SUB)])


@functools.partial(
    pl.kernel,
    out_type=jax.ShapeDtypeStruct((NC, N_PAD, D), jnp.float32),
    mesh=_mesh,
    scratch_types=[
        [pltpu.VMEM((CM, D), jnp.float32) for _ in range(NBM)],
        [pltpu.VMEM((CM,), jnp.int32) for _ in range(NBM)],
        [pltpu.VMEM((CM,), jnp.int32) for _ in range(NBM)],
        [pltpu.VMEM((CM,), jnp.float32) for _ in range(NBM)],
        [pltpu.SemaphoreType.DMA for _ in range(NBM)],
        [pltpu.SemaphoreType.DMA for _ in range(NBM)],
        [pltpu.SemaphoreType.DMA for _ in range(NBM)],
        [pltpu.SemaphoreType.DMA for _ in range(NBM)],
        [pltpu.SemaphoreType.DMA for _ in range(NBM)],
        pltpu.VMEM_SHARED((N_PAD, D), jnp.float32),
    ],
    compiler_params=_sc_params,
)
def _sc_msg(g_hbm, src2d, dst2d, w2d, acc_out,
            rows, srcb, dstb, wb, gsem, ssem, srcsem, dstsem, wsem, acc_sh):
    cid = lax.axis_index("c")
    sid = lax.axis_index("s")
    zofs = sid * ROWS_PER_SUB
    on0 = cid == 0
    km = jnp.where(on0, KM0, KM1)
    kg = jnp.where(on0, KG0, KG1)
    base = jnp.where(on0, sid * KM0, NS * KM0 + sid * KM1)

    def zrow(i, carry):
        for j in range(D // 16):
            rows[0][i, pl.ds(j * 16, 16)] = jnp.zeros((16,), jnp.float32)
        return carry
    lax.fori_loop(0, CM, zrow, 0)
    for i in range(ROWS_PER_SUB // CM):
        pltpu.sync_copy(rows[0], acc_sh.at[pl.ds(zofs + i * CM, CM)])

    for b in range(NBM):
        pltpu.sync_copy(src2d.at[base + b], srcb[b])
        pltpu.sync_copy(w2d.at[base + b], wb[b])
    for b in range(NBM - 1):
        pltpu.sync_copy(dst2d.at[base + b], dstb[b])
    plsc.subcore_barrier()
    for b in range(NBM - 1):
        pltpu.async_copy(g_hbm.at[srcb[b]], rows[b], gsem[b])

    def mul_chunk(b, ww):
        def mul_e(e, c2):
            splat = plsc.load_gather(
                ww, [jnp.full((16,), 0, jnp.int32) + e])
            for j in range(D // 16):
                sl = pl.ds(j * 16, 16)
                rows[b][e, sl] = rows[b][e, sl] * splat
            return c2
        lax.fori_loop(0, CM, mul_e, 0)

    def group(g, carry):
        for b in range(NBM):
            k = g * NBM + b
            b2 = (b + 2) % NBM
            pltpu.make_async_copy(g_hbm.at[srcb[b]], rows[b], gsem[b]).wait()
            mul_chunk(b, wb[b])

            @pl.when(k >= 2)
            def _():
                pltpu.make_async_copy(
                    dst2d.at[base], dstb[b], dstsem[b]).wait()
            pltpu.async_copy(rows[b], acc_sh.at[dstb[b]], ssem[b], add=True)

            @pl.when(k >= 1)
            def _():
                pltpu.make_async_copy(
                    rows[b2], acc_sh.at[dstb[b2]], ssem[b2]).wait()

            @pl.when(k + 2 < km)
            def _():
                @pl.when(k >= 1)
                def _():
                    pltpu.make_async_copy(
                        src2d.at[base], srcb[b2], srcsem[b2]).wait()
                    pltpu.make_async_copy(
                        w2d.at[base], wb[b2], wsem[b2]).wait()
                pltpu.async_copy(g_hbm.at[srcb[b2]], rows[b2], gsem[b2])
                pltpu.async_copy(
                    dst2d.at[base + k + 2], dstb[b2], dstsem[b2])

            @pl.when(k + 3 < km)
            def _():
                pltpu.async_copy(src2d.at[base + k + 3], srcb[b], srcsem[b])
                pltpu.async_copy(w2d.at[base + k + 3], wb[b], wsem[b])
        return carry

    lax.fori_loop(0, kg, group, 0)

    for b, koff in ((0, 2), (1, 1)):
        kt = km - koff
        pltpu.make_async_copy(g_hbm.at[srcb[b]], rows[b], gsem[b]).wait()
        mul_chunk(b, wb[b])
        pltpu.make_async_copy(dst2d.at[base], dstb[b], dstsem[b]).wait()
        pltpu.async_copy(rows[b], acc_sh.at[dstb[b]], ssem[b], add=True)
        b2 = (b + 2) % NBM
        pltpu.make_async_copy(rows[b2], acc_sh.at[dstb[b2]], ssem[b2]).wait()
    pltpu.make_async_copy(rows[1], acc_sh.at[dstb[1]], ssem[1]).wait()
    plsc.subcore_barrier()
    pltpu.sync_copy(acc_sh.at[pl.ds(zofs, ROWS_PER_SUB)],
                    acc_out.at[cid, pl.ds(zofs, ROWS_PER_SUB)])


def _tc_dis_body(deg_ref, out_ref):
    out_ref[...] = lax.rsqrt(1.0 + deg_ref[0:1, :] + deg_ref[1:2, :])


def _tc_dis(deg2):
    return pl.pallas_call(
        _tc_dis_body,
        out_shape=jax.ShapeDtypeStruct((1, N_PAD), jnp.float32),
    )(deg2)


_MM_BLK = 640


def _tc_mm_body(x_ref, w_ref, dis_ref, g_ref):
    mm = jnp.dot(x_ref[...], w_ref[...], preferred_element_type=jnp.float32)
    g_ref[...] = mm * dis_ref[...]


def _tc_mm(x_p, W, dis_col):
    grid = (N_PAD // _MM_BLK,)
    return pl.pallas_call(
        _tc_mm_body,
        grid=grid,
        in_specs=[
            pl.BlockSpec((_MM_BLK, D), lambda i: (i, 0)),
            pl.BlockSpec((D, D), lambda i: (0, 0)),
            pl.BlockSpec((_MM_BLK, 1), lambda i: (i, 0)),
        ],
        out_specs=pl.BlockSpec((_MM_BLK, D), lambda i: (i, 0)),
        out_shape=jax.ShapeDtypeStruct((N_PAD, D), jnp.float32),
    )(x_p, W, dis_col)


def _tc_fin_body(a0_ref, a1_ref, g_ref, dis_ref, b_ref, pa_ref, out_ref):
    o = dis_ref[...] * (a0_ref[...] + a1_ref[...] + g_ref[...]) + b_ref[...]
    out_ref[...] = jnp.where(o >= 0.0, o, pa_ref[...] * o)


def _tc_fin(acc0, acc1, g, dis_col, b2, pa2):
    grid = (N_PAD // _MM_BLK,)
    return pl.pallas_call(
        _tc_fin_body,
        grid=grid,
        in_specs=[
            pl.BlockSpec((_MM_BLK, D), lambda i: (i, 0)),
            pl.BlockSpec((_MM_BLK, D), lambda i: (i, 0)),
            pl.BlockSpec((_MM_BLK, D), lambda i: (i, 0)),
            pl.BlockSpec((_MM_BLK, 1), lambda i: (i, 0)),
            pl.BlockSpec((1, D), lambda i: (0, 0)),
            pl.BlockSpec((1, D), lambda i: (0, 0)),
        ],
        out_specs=pl.BlockSpec((_MM_BLK, D), lambda i: (i, 0)),
        out_shape=jax.ShapeDtypeStruct((N_PAD, D), jnp.float32),
    )(acc0, acc1, g, dis_col, b2, pa2)


@jax.jit
def kernel(x, edge_index, edge_weight, W, b, prelu_a):
    src2m = edge_index[0].reshape(RM, CM)
    dst2m = edge_index[1].reshape(RM, CM)
    w2m = edge_weight.reshape(RM, CM)
    dst2d = edge_index[1].reshape(RD, CD)
    w2d = edge_weight.reshape(RD, CD)
    x_p = jnp.concatenate(
        [x, jnp.zeros((N_PAD - N, D), jnp.float32)], axis=0)

    deg2 = _sc_deg(dst2d, w2d)
    dis_col = _tc_dis(deg2).reshape(N_PAD, 1)
    g = _tc_mm(x_p, W, dis_col)
    acc2 = _sc_msg(g, src2m, dst2m, w2m)
    out = _tc_fin(acc2[0], acc2[1], g, dis_col,
                  b.reshape(1, D), prelu_a.reshape(1, D))
    return out[:N]


# speedup vs baseline: 33.2186x; 1.1360x over previous
import functools

import jax
import jax.numpy as jnp
from jax import lax
from jax.experimental import pallas as pl
from jax.experimental.pallas import tpu as pltpu
from jax.experimental.pallas import tpu_sc as plsc

N = 10000
E = 320000
D = 128

NC = 2
NS = 16
NW = NC * NS

N_PAD = 10240
ROWS_PER_SUB = N_PAD // NS

CD = 125
KD = E // (NW * CD)
RD = E // CD

CM = 80
NBM = 3
KM = E // (NW * CM)
KGM = (KM - 2) // NBM
EPW = E // NW

_mesh = plsc.VectorSubcoreMesh(core_axis_name="c", subcore_axis_name="s")
_sc_params = pltpu.CompilerParams(needs_layout_passes=False)


@functools.partial(
    pl.kernel,
    out_type=jax.ShapeDtypeStruct((NC, N_PAD), jnp.float32),
    mesh=_mesh,
    scratch_types=[
        pltpu.VMEM((KD, CD), jnp.int32),
        pltpu.VMEM((KD, CD), jnp.float32),
        pltpu.VMEM((ROWS_PER_SUB,), jnp.float32),
        pltpu.VMEM_SHARED((N_PAD,), jnp.float32),
    ],
    compiler_params=_sc_params,
)
def _sc_deg(dst2d, w2d, deg_out, dst_all, w_all, zline_v, deg_sh):
    cid = lax.axis_index("c")
    sid = lax.axis_index("s")
    wid = sid * NC + cid
    zofs = sid * ROWS_PER_SUB

    def zrow(i, carry):
        zline_v[pl.ds(i * 16, 16)] = jnp.zeros((16,), jnp.float32)
        return carry
    lax.fori_loop(0, ROWS_PER_SUB // 16, zrow, 0)
    pltpu.sync_copy(zline_v, deg_sh.at[pl.ds(zofs, ROWS_PER_SUB)])
    pltpu.sync_copy(dst2d.at[pl.ds(wid * KD, KD)], dst_all)
    pltpu.sync_copy(w2d.at[pl.ds(wid * KD, KD)], w_all)
    plsc.subcore_barrier()

    def chunk(k, carry):
        pltpu.sync_copy(w_all.at[k], deg_sh.at[dst_all.at[k]], add=True)
        return carry

    lax.fori_loop(0, KD, chunk, 0)
    plsc.subcore_barrier()
    pltpu.sync_copy(deg_sh.at[pl.ds(zofs, ROWS_PER_SUB)],
                    deg_out.at[cid, pl.ds(zofs, ROWS_PER_SUB)])


@functools.partial(
    pl.kernel,
    out_type=jax.ShapeDtypeStruct((NC, N_PAD, D), jnp.float32),
    mesh=_mesh,
    scratch_types=[
        [pltpu.VMEM((CM, D), jnp.float32) for _ in range(NBM)],
        [pltpu.VMEM((CM,), jnp.int32) for _ in range(NBM)],
        [pltpu.VMEM((CM,), jnp.int32) for _ in range(NBM)],
        [pltpu.VMEM((CM,), jnp.float32) for _ in range(NBM)],
        [pltpu.SemaphoreType.DMA for _ in range(NBM)],
        [pltpu.SemaphoreType.DMA for _ in range(NBM)],
        [pltpu.SemaphoreType.DMA for _ in range(NBM)],
        [pltpu.SemaphoreType.DMA for _ in range(NBM)],
        [pltpu.SemaphoreType.DMA for _ in range(NBM)],
        pltpu.VMEM_SHARED((N_PAD, D), jnp.float32),
    ],
    compiler_params=_sc_params,
)
def _sc_msg(g_hbm, src1, dst1, w1, acc_out,
            rows, srcb, dstb, wb, gsem, ssem, srcsem, dstsem, wsem, acc_sh):
    cid = lax.axis_index("c")
    sid = lax.axis_index("s")
    zofs = sid * ROWS_PER_SUB
    wid = sid * NC + cid
    base = wid * EPW

    def zrow(i, carry):
        for j in range(D // 16):
            rows[0][i, pl.ds(j * 16, 16)] = jnp.zeros((16,), jnp.float32)
        return carry
    lax.fori_loop(0, CM, zrow, 0)
    for i in range(ROWS_PER_SUB // CM):
        pltpu.sync_copy(rows[0], acc_sh.at[pl.ds(zofs + i * CM, CM)])

    for b in range(NBM):
        pltpu.sync_copy(src1.at[pl.ds(base + b * CM, CM)], srcb[b])
        pltpu.sync_copy(w1.at[pl.ds(base + b * CM, CM)], wb[b])
    for b in range(NBM - 1):
        pltpu.sync_copy(dst1.at[pl.ds(base + b * CM, CM)], dstb[b])
    plsc.subcore_barrier()
    for b in range(NBM - 1):
        pltpu.async_copy(g_hbm.at[srcb[b]], rows[b], gsem[b])

    def mul_chunk(b, ww):
        def mul_e(e, c2):
            splat = plsc.load_gather(
                ww, [jnp.full((16,), 0, jnp.int32) + e])
            for j in range(D // 16):
                sl = pl.ds(j * 16, 16)
                rows[b][e, sl] = rows[b][e, sl] * splat
            return c2
        lax.fori_loop(0, CM, mul_e, 0)

    def group(g, carry):
        for b in range(NBM):
            k = g * NBM + b
            b2 = (b + 2) % NBM
            pltpu.make_async_copy(g_hbm.at[srcb[b]], rows[b], gsem[b]).wait()
            mul_chunk(b, wb[b])

            @pl.when(k >= 2)
            def _():
                pltpu.make_async_copy(
                    dst1.at[pl.ds(base, CM)], dstb[b], dstsem[b]).wait()
            pltpu.async_copy(rows[b], acc_sh.at[dstb[b]], ssem[b], add=True)

            @pl.when(k >= 1)
            def _():
                pltpu.make_async_copy(
                    rows[b2], acc_sh.at[dstb[b2]], ssem[b2]).wait()

            @pl.when(k + 2 < KM)
            def _():
                @pl.when(k >= 1)
                def _():
                    pltpu.make_async_copy(
                        src1.at[pl.ds(base, CM)], srcb[b2], srcsem[b2]).wait()
                    pltpu.make_async_copy(
                        w1.at[pl.ds(base, CM)], wb[b2], wsem[b2]).wait()
                pltpu.async_copy(g_hbm.at[srcb[b2]], rows[b2], gsem[b2])
                pltpu.async_copy(
                    dst1.at[pl.ds(base + (k + 2) * CM, CM)],
                    dstb[b2], dstsem[b2])

            @pl.when(k + 3 < KM)
            def _():
                pltpu.async_copy(
                    src1.at[pl.ds(base + (k + 3) * CM, CM)], srcb[b], srcsem[b])
                pltpu.async_copy(
                    w1.at[pl.ds(base + (k + 3) * CM, CM)], wb[b], wsem[b])
        return carry

    lax.fori_loop(0, KGM, group, 0)

    for b in (0, 1):
        pltpu.make_async_copy(g_hbm.at[srcb[b]], rows[b], gsem[b]).wait()
        mul_chunk(b, wb[b])
        pltpu.make_async_copy(
            dst1.at[pl.ds(base, CM)], dstb[b], dstsem[b]).wait()
        pltpu.async_copy(rows[b], acc_sh.at[dstb[b]], ssem[b], add=True)
        b2 = (b + 2) % NBM
        pltpu.make_async_copy(rows[b2], acc_sh.at[dstb[b2]], ssem[b2]).wait()
    pltpu.make_async_copy(rows[1], acc_sh.at[dstb[1]], ssem[1]).wait()
    plsc.subcore_barrier()
    pltpu.sync_copy(acc_sh.at[pl.ds(zofs, ROWS_PER_SUB)],
                    acc_out.at[cid, pl.ds(zofs, ROWS_PER_SUB)])


def _tc_dis_body(deg_ref, out_ref):
    out_ref[...] = lax.rsqrt(1.0 + deg_ref[0:1, :] + deg_ref[1:2, :])


def _tc_dis(deg2):
    return pl.pallas_call(
        _tc_dis_body,
        out_shape=jax.ShapeDtypeStruct((1, N_PAD), jnp.float32),
    )(deg2)


_MM_BLK = 640


def _tc_mm_body(x_ref, w_ref, dis_ref, g_ref):
    mm = jnp.dot(x_ref[...], w_ref[...], preferred_element_type=jnp.float32)
    g_ref[...] = mm * dis_ref[...]


def _tc_mm(x_p, W, dis_col):
    grid = (N_PAD // _MM_BLK,)
    return pl.pallas_call(
        _tc_mm_body,
        grid=grid,
        in_specs=[
            pl.BlockSpec((_MM_BLK, D), lambda i: (i, 0)),
            pl.BlockSpec((D, D), lambda i: (0, 0)),
            pl.BlockSpec((_MM_BLK, 1), lambda i: (i, 0)),
        ],
        out_specs=pl.BlockSpec((_MM_BLK, D), lambda i: (i, 0)),
        out_shape=jax.ShapeDtypeStruct((N_PAD, D), jnp.float32),
    )(x_p, W, dis_col)


_FIN_BLK = 400


def _tc_fin_body(acc_ref, g_ref, dis_ref, b_ref, pa_ref, out_ref):
    o = dis_ref[...] * (acc_ref[0] + acc_ref[1] + g_ref[...]) + b_ref[...]
    out_ref[...] = jnp.where(o >= 0.0, o, pa_ref[...] * o)


def _tc_fin(acc2, g, dis_col, b2, pa2):
    grid = (N // _FIN_BLK,)
    return pl.pallas_call(
        _tc_fin_body,
        grid=grid,
        in_specs=[
            pl.BlockSpec((NC, _FIN_BLK, D), lambda i: (0, i, 0)),
            pl.BlockSpec((_FIN_BLK, D), lambda i: (i, 0)),
            pl.BlockSpec((_FIN_BLK, 1), lambda i: (i, 0)),
            pl.BlockSpec((1, D), lambda i: (0, 0)),
            pl.BlockSpec((1, D), lambda i: (0, 0)),
        ],
        out_specs=pl.BlockSpec((_FIN_BLK, D), lambda i: (i, 0)),
        out_shape=jax.ShapeDtypeStruct((N, D), jnp.float32),
    )(acc2, g, dis_col, b2, pa2)


@jax.jit
def kernel(x, edge_index, edge_weight, W, b, prelu_a):
    src1 = edge_index[0]
    dst1 = edge_index[1]
    dst2d = dst1.reshape(RD, CD)
    w2d = edge_weight.reshape(RD, CD)
    x_p = jnp.concatenate(
        [x, jnp.zeros((N_PAD - N, D), jnp.float32)], axis=0)

    deg2 = _sc_deg(dst2d, w2d)
    dis_col = _tc_dis(deg2).reshape(N_PAD, 1)
    g = _tc_mm(x_p, W, dis_col)
    acc2 = _sc_msg(g, src1, dst1, edge_weight)
    return _tc_fin(acc2, g[:N], dis_col[:N],
                   b.reshape(1, D), prelu_a.reshape(1, D))


# speedup vs baseline: 34.2257x; 1.0303x over previous
import functools

import jax
import jax.numpy as jnp
from jax import lax
from jax.experimental import pallas as pl
from jax.experimental.pallas import tpu as pltpu
from jax.experimental.pallas import tpu_sc as plsc

N = 10000
E = 320000
D = 128

NC = 2
NS = 16
NW = NC * NS

N_PAD = 10240
ROWS_PER_SUB = N_PAD // NS

CD = 125
KD = E // (NW * CD)
RD = E // CD

CM = 80
NBM = 3
KM = E // (NW * CM)
KGM = (KM - 2) // NBM
EPW = E // NW

_mesh = plsc.VectorSubcoreMesh(core_axis_name="c", subcore_axis_name="s")
_sc_params = pltpu.CompilerParams(needs_layout_passes=False)


@functools.partial(
    pl.kernel,
    out_type=jax.ShapeDtypeStruct((NC, N_PAD), jnp.float32),
    mesh=_mesh,
    scratch_types=[
        pltpu.VMEM((KD, CD), jnp.int32),
        pltpu.VMEM((KD, CD), jnp.float32),
        pltpu.VMEM((ROWS_PER_SUB,), jnp.float32),
        pltpu.VMEM_SHARED((N_PAD,), jnp.float32),
    ],
    compiler_params=_sc_params,
)
def _sc_deg(dst2d, w2d, deg_out, dst_all, w_all, zline_v, deg_sh):
    cid = lax.axis_index("c")
    sid = lax.axis_index("s")
    wid = sid * NC + cid
    zofs = sid * ROWS_PER_SUB

    def zrow(i, carry):
        zline_v[pl.ds(i * 16, 16)] = jnp.zeros((16,), jnp.float32)
        return carry
    lax.fori_loop(0, ROWS_PER_SUB // 16, zrow, 0)
    pltpu.sync_copy(zline_v, deg_sh.at[pl.ds(zofs, ROWS_PER_SUB)])
    pltpu.sync_copy(dst2d.at[pl.ds(wid * KD, KD)], dst_all)
    pltpu.sync_copy(w2d.at[pl.ds(wid * KD, KD)], w_all)
    plsc.subcore_barrier()

    def chunk(k, carry):
        pltpu.sync_copy(w_all.at[k], deg_sh.at[dst_all.at[k]], add=True)
        return carry

    lax.fori_loop(0, KD, chunk, 0)
    plsc.subcore_barrier()
    pltpu.sync_copy(deg_sh.at[pl.ds(zofs, ROWS_PER_SUB)],
                    deg_out.at[cid, pl.ds(zofs, ROWS_PER_SUB)])


@functools.partial(
    pl.kernel,
    out_type=jax.ShapeDtypeStruct((NC, N_PAD, D), jnp.float32),
    mesh=_mesh,
    scratch_types=[
        [pltpu.VMEM((CM, D), jnp.float32) for _ in range(NBM)],
        [pltpu.VMEM((CM,), jnp.int32) for _ in range(NBM)],
        [pltpu.VMEM((CM,), jnp.int32) for _ in range(NBM)],
        [pltpu.VMEM((CM,), jnp.float32) for _ in range(NBM)],
        [pltpu.SemaphoreType.DMA for _ in range(NBM)],
        [pltpu.SemaphoreType.DMA for _ in range(NBM)],
        [pltpu.SemaphoreType.DMA for _ in range(NBM)],
        [pltpu.SemaphoreType.DMA for _ in range(NBM)],
        [pltpu.SemaphoreType.DMA for _ in range(NBM)],
        pltpu.VMEM((N_PAD,), jnp.float32),
        pltpu.VMEM_SHARED((N_PAD, D), jnp.float32),
    ],
    compiler_params=_sc_params,
)
def _sc_msg(g_hbm, src1, dst1, w1, dis1, acc_out,
            rows, srcb, dstb, wb, gsem, ssem, srcsem, dstsem, wsem,
            dis_v, acc_sh):
    cid = lax.axis_index("c")
    sid = lax.axis_index("s")
    zofs = sid * ROWS_PER_SUB
    wid = sid * NC + cid
    base = wid * EPW

    def zrow(i, carry):
        for j in range(D // 16):
            rows[0][i, pl.ds(j * 16, 16)] = jnp.zeros((16,), jnp.float32)
        return carry
    lax.fori_loop(0, CM, zrow, 0)
    for i in range(ROWS_PER_SUB // CM):
        pltpu.sync_copy(rows[0], acc_sh.at[pl.ds(zofs + i * CM, CM)])

    pltpu.sync_copy(dis1, dis_v)
    for b in range(NBM):
        pltpu.sync_copy(src1.at[pl.ds(base + b * CM, CM)], srcb[b])
        pltpu.sync_copy(w1.at[pl.ds(base + b * CM, CM)], wb[b])
    for b in range(NBM - 1):
        pltpu.sync_copy(dst1.at[pl.ds(base + b * CM, CM)], dstb[b])
    plsc.subcore_barrier()
    for b in range(NBM - 1):
        pltpu.async_copy(g_hbm.at[srcb[b]], rows[b], gsem[b])

    def mul_chunk(b, ww):
        for i in range(CM // 16):
            sl = pl.ds(i * 16, 16)
            d16 = plsc.load_gather(dis_v, [srcb[b][sl]])
            ww[sl] = ww[sl] * d16

        def mul_e(e, c2):
            splat = plsc.load_gather(
                ww, [jnp.full((16,), 0, jnp.int32) + e])
            for j in range(D // 16):
                sl = pl.ds(j * 16, 16)
                rows[b][e, sl] = rows[b][e, sl] * splat
            return c2
        lax.fori_loop(0, CM, mul_e, 0)

    def group(g, carry):
        for b in range(NBM):
            k = g * NBM + b
            b2 = (b + 2) % NBM
            pltpu.make_async_copy(g_hbm.at[srcb[b]], rows[b], gsem[b]).wait()
            mul_chunk(b, wb[b])

            @pl.when(k >= 2)
            def _():
                pltpu.make_async_copy(
                    dst1.at[pl.ds(base, CM)], dstb[b], dstsem[b]).wait()
            pltpu.async_copy(rows[b], acc_sh.at[dstb[b]], ssem[b], add=True)

            @pl.when(k >= 1)
            def _():
                pltpu.make_async_copy(
                    rows[b2], acc_sh.at[dstb[b2]], ssem[b2]).wait()

            @pl.when(k + 2 < KM)
            def _():
                @pl.when(k >= 1)
                def _():
                    pltpu.make_async_copy(
                        src1.at[pl.ds(base, CM)], srcb[b2], srcsem[b2]).wait()
                    pltpu.make_async_copy(
                        w1.at[pl.ds(base, CM)], wb[b2], wsem[b2]).wait()
                pltpu.async_copy(g_hbm.at[srcb[b2]], rows[b2], gsem[b2])
                pltpu.async_copy(
                    dst1.at[pl.ds(base + (k + 2) * CM, CM)],
                    dstb[b2], dstsem[b2])

            @pl.when(k + 3 < KM)
            def _():
                pltpu.async_copy(
                    src1.at[pl.ds(base + (k + 3) * CM, CM)], srcb[b], srcsem[b])
                pltpu.async_copy(
                    w1.at[pl.ds(base + (k + 3) * CM, CM)], wb[b], wsem[b])
        return carry

    lax.fori_loop(0, KGM, group, 0)

    for b in (0, 1):
        pltpu.make_async_copy(g_hbm.at[srcb[b]], rows[b], gsem[b]).wait()
        mul_chunk(b, wb[b])
        pltpu.make_async_copy(
            dst1.at[pl.ds(base, CM)], dstb[b], dstsem[b]).wait()
        pltpu.async_copy(rows[b], acc_sh.at[dstb[b]], ssem[b], add=True)
        b2 = (b + 2) % NBM
        pltpu.make_async_copy(rows[b2], acc_sh.at[dstb[b2]], ssem[b2]).wait()
    pltpu.make_async_copy(rows[1], acc_sh.at[dstb[1]], ssem[1]).wait()
    plsc.subcore_barrier()
    pltpu.sync_copy(acc_sh.at[pl.ds(zofs, ROWS_PER_SUB)],
                    acc_out.at[cid, pl.ds(zofs, ROWS_PER_SUB)])


def _tc_dis_body(deg_ref, out_ref):
    out_ref[...] = lax.rsqrt(1.0 + deg_ref[0:1, :] + deg_ref[1:2, :])


def _tc_dis(deg2):
    return pl.pallas_call(
        _tc_dis_body,
        out_shape=jax.ShapeDtypeStruct((1, N_PAD), jnp.float32),
    )(deg2)


_MM_BLK = 640


def _tc_mm_body(x_ref, w_ref, h_ref):
    h_ref[...] = jnp.dot(
        x_ref[...], w_ref[...], preferred_element_type=jnp.float32)


def _tc_mm(x_p, W):
    grid = (N_PAD // _MM_BLK,)
    return pl.pallas_call(
        _tc_mm_body,
        grid=grid,
        in_specs=[
            pl.BlockSpec((_MM_BLK, D), lambda i: (i, 0)),
            pl.BlockSpec((D, D), lambda i: (0, 0)),
        ],
        out_specs=pl.BlockSpec((_MM_BLK, D), lambda i: (i, 0)),
        out_shape=jax.ShapeDtypeStruct((N_PAD, D), jnp.float32),
    )(x_p, W)


_FIN_BLK = 400


def _tc_fin_body(acc_ref, h_ref, dis_ref, b_ref, pa_ref, out_ref):
    dis = dis_ref[...]
    o = dis * (acc_ref[0] + acc_ref[1] + dis * h_ref[...]) + b_ref[...]
    out_ref[...] = jnp.where(o >= 0.0, o, pa_ref[...] * o)


def _tc_fin(acc2, g, dis_col, b2, pa2):
    grid = (N // _FIN_BLK,)
    return pl.pallas_call(
        _tc_fin_body,
        grid=grid,
        in_specs=[
            pl.BlockSpec((NC, _FIN_BLK, D), lambda i: (0, i, 0)),
            pl.BlockSpec((_FIN_BLK, D), lambda i: (i, 0)),
            pl.BlockSpec((_FIN_BLK, 1), lambda i: (i, 0)),
            pl.BlockSpec((1, D), lambda i: (0, 0)),
            pl.BlockSpec((1, D), lambda i: (0, 0)),
        ],
        out_specs=pl.BlockSpec((_FIN_BLK, D), lambda i: (i, 0)),
        out_shape=jax.ShapeDtypeStruct((N, D), jnp.float32),
    )(acc2, g, dis_col, b2, pa2)


@jax.jit
def kernel(x, edge_index, edge_weight, W, b, prelu_a):
    src1 = edge_index[0]
    dst1 = edge_index[1]
    dst2d = dst1.reshape(RD, CD)
    w2d = edge_weight.reshape(RD, CD)
    x_p = jnp.concatenate(
        [x, jnp.zeros((N_PAD - N, D), jnp.float32)], axis=0)

    h = _tc_mm(x_p, W)
    deg2 = _sc_deg(dst2d, w2d)
    dis_row = _tc_dis(deg2)
    acc2 = _sc_msg(h, src1, dst1, edge_weight, dis_row.reshape(N_PAD))
    return _tc_fin(acc2, h[:N], dis_row.reshape(N_PAD, 1)[:N],
                   b.reshape(1, D), prelu_a.reshape(1, D))


# speedup vs baseline: 35.6722x; 1.0423x over previous
import functools

import jax
import jax.numpy as jnp
from jax import lax
from jax.experimental import pallas as pl
from jax.experimental.pallas import tpu as pltpu
from jax.experimental.pallas import tpu_sc as plsc

N = 10000
E = 320000
D = 128

NC = 2
NS = 16
NW = NC * NS

N_PAD = 10240
ROWS_PER_SUB = N_PAD // NS

CM = 80
NBM = 3
KM = E // (NW * CM)
KGM = (KM - 2) // NBM
EPW = E // NW

_mesh = plsc.VectorSubcoreMesh(core_axis_name="c", subcore_axis_name="s")
_sc_params = pltpu.CompilerParams(needs_layout_passes=False)


NBD = 4
KD2 = E // (NW * CM)


@functools.partial(
    pl.kernel,
    out_type=jax.ShapeDtypeStruct((NC, N_PAD), jnp.float32),
    mesh=_mesh,
    scratch_types=[
        [pltpu.VMEM((CM,), jnp.int32) for _ in range(NBD)],
        [pltpu.VMEM((CM,), jnp.float32) for _ in range(NBD)],
        [pltpu.SemaphoreType.DMA for _ in range(NBD)],
        [pltpu.SemaphoreType.DMA for _ in range(NBD)],
        [pltpu.SemaphoreType.DMA for _ in range(NBD)],
        pltpu.VMEM((ROWS_PER_SUB,), jnp.float32),
        pltpu.VMEM_SHARED((N_PAD,), jnp.float32),
    ],
    compiler_params=_sc_params,
)
def _sc_deg(dst1, w1, deg_out, dstb, wb, dsem, wsem, ssem, zline_v, deg_sh):
    cid = lax.axis_index("c")
    sid = lax.axis_index("s")
    wid = sid * NC + cid
    zofs = sid * ROWS_PER_SUB
    base = wid * EPW

    def zrow(i, carry):
        zline_v[pl.ds(i * 16, 16)] = jnp.zeros((16,), jnp.float32)
        return carry
    lax.fori_loop(0, ROWS_PER_SUB // 16, zrow, 0)
    pltpu.sync_copy(zline_v, deg_sh.at[pl.ds(zofs, ROWS_PER_SUB)])
    for b in range(2):
        pltpu.sync_copy(dst1.at[pl.ds(base + b * CM, CM)], dstb[b])
        pltpu.sync_copy(w1.at[pl.ds(base + b * CM, CM)], wb[b])
    plsc.subcore_barrier()

    def group(g, carry):
        for b in range(NBD):
            j = g * NBD + b
            b3 = (b + 2) % NBD
            @pl.when(j >= 2)
            def _():
                pltpu.make_async_copy(
                    dst1.at[pl.ds(base, CM)], dstb[b], dsem[b]).wait()
                pltpu.make_async_copy(
                    w1.at[pl.ds(base, CM)], wb[b], wsem[b]).wait()
            pltpu.async_copy(wb[b], deg_sh.at[dstb[b]], ssem[b], add=True)

            @pl.when(j >= 2)
            def _():
                pltpu.make_async_copy(
                    wb[b3], deg_sh.at[dstb[b3]], ssem[b3]).wait()

            @pl.when(j + 2 < KD2)
            def _():
                pltpu.async_copy(
                    dst1.at[pl.ds(base + (j + 2) * CM, CM)], dstb[b3],
                    dsem[b3])
                pltpu.async_copy(
                    w1.at[pl.ds(base + (j + 2) * CM, CM)], wb[b3], wsem[b3])
        return carry

    lax.fori_loop(0, (KD2 - 1) // NBD, group, 0)
    pltpu.make_async_copy(dst1.at[pl.ds(base, CM)], dstb[0], dsem[0]).wait()
    pltpu.make_async_copy(w1.at[pl.ds(base, CM)], wb[0], wsem[0]).wait()
    pltpu.async_copy(wb[0], deg_sh.at[dstb[0]], ssem[0], add=True)
    for b in (2, 3, 0):
        pltpu.make_async_copy(wb[b], deg_sh.at[dstb[b]], ssem[b]).wait()
    plsc.subcore_barrier()
    pltpu.sync_copy(deg_sh.at[pl.ds(zofs, ROWS_PER_SUB)],
                    deg_out.at[cid, pl.ds(zofs, ROWS_PER_SUB)])


@functools.partial(
    pl.kernel,
    out_type=jax.ShapeDtypeStruct((NC, N_PAD, D), jnp.float32),
    mesh=_mesh,
    scratch_types=[
        [pltpu.VMEM((CM, D), jnp.float32) for _ in range(NBM)],
        [pltpu.VMEM((CM,), jnp.int32) for _ in range(NBM)],
        [pltpu.VMEM((CM,), jnp.int32) for _ in range(NBM)],
        [pltpu.VMEM((CM,), jnp.float32) for _ in range(NBM)],
        [pltpu.SemaphoreType.DMA for _ in range(NBM)],
        [pltpu.SemaphoreType.DMA for _ in range(NBM)],
        [pltpu.SemaphoreType.DMA for _ in range(NBM)],
        [pltpu.SemaphoreType.DMA for _ in range(NBM)],
        [pltpu.SemaphoreType.DMA for _ in range(NBM)],
        pltpu.VMEM((N_PAD,), jnp.float32),
        pltpu.VMEM_SHARED((N_PAD, D), jnp.float32),
    ],
    compiler_params=_sc_params,
)
def _sc_msg(g_hbm, src1, dst1, w1, dis1, acc_out,
            rows, srcb, dstb, wb, gsem, ssem, srcsem, dstsem, wsem,
            dis_v, acc_sh):
    cid = lax.axis_index("c")
    sid = lax.axis_index("s")
    zofs = sid * ROWS_PER_SUB
    wid = sid * NC + cid
    base = wid * EPW

    def zrow(i, carry):
        for j in range(D // 16):
            rows[0][i, pl.ds(j * 16, 16)] = jnp.zeros((16,), jnp.float32)
        return carry
    lax.fori_loop(0, CM, zrow, 0)
    for i in range(ROWS_PER_SUB // CM):
        pltpu.sync_copy(rows[0], acc_sh.at[pl.ds(zofs + i * CM, CM)])

    pltpu.sync_copy(dis1, dis_v)
    for b in range(NBM):
        pltpu.sync_copy(src1.at[pl.ds(base + b * CM, CM)], srcb[b])
        pltpu.sync_copy(w1.at[pl.ds(base + b * CM, CM)], wb[b])
    for b in range(NBM - 1):
        pltpu.sync_copy(dst1.at[pl.ds(base + b * CM, CM)], dstb[b])
    plsc.subcore_barrier()
    for b in range(NBM - 1):
        pltpu.async_copy(g_hbm.at[srcb[b]], rows[b], gsem[b])

    def mul_chunk(b, ww):
        for i in range(CM // 16):
            sl = pl.ds(i * 16, 16)
            d16 = plsc.load_gather(dis_v, [srcb[b][sl]])
            ww[sl] = ww[sl] * d16

        @plsc.parallel_loop(0, CM, 1, unroll=2)
        def _(e):
            splat = plsc.load_gather(
                ww, [jnp.full((16,), 0, jnp.int32) + e])
            for j in range(D // 16):
                sl = pl.ds(j * 16, 16)
                rows[b][e, sl] = rows[b][e, sl] * splat

    def group(g, carry):
        for b in range(NBM):
            k = g * NBM + b
            b2 = (b + 2) % NBM
            pltpu.make_async_copy(g_hbm.at[srcb[b]], rows[b], gsem[b]).wait()
            mul_chunk(b, wb[b])

            @pl.when(k >= 2)
            def _():
                pltpu.make_async_copy(
                    dst1.at[pl.ds(base, CM)], dstb[b], dstsem[b]).wait()
            pltpu.async_copy(rows[b], acc_sh.at[dstb[b]], ssem[b], add=True)

            @pl.when(k >= 1)
            def _():
                pltpu.make_async_copy(
                    rows[b2], acc_sh.at[dstb[b2]], ssem[b2]).wait()

            @pl.when(k + 2 < KM)
            def _():
                @pl.when(k >= 1)
                def _():
                    pltpu.make_async_copy(
                        src1.at[pl.ds(base, CM)], srcb[b2], srcsem[b2]).wait()
                    pltpu.make_async_copy(
                        w1.at[pl.ds(base, CM)], wb[b2], wsem[b2]).wait()
                pltpu.async_copy(g_hbm.at[srcb[b2]], rows[b2], gsem[b2])
                pltpu.async_copy(
                    dst1.at[pl.ds(base + (k + 2) * CM, CM)],
                    dstb[b2], dstsem[b2])

            @pl.when(k + 3 < KM)
            def _():
                pltpu.async_copy(
                    src1.at[pl.ds(base + (k + 3) * CM, CM)], srcb[b], srcsem[b])
                pltpu.async_copy(
                    w1.at[pl.ds(base + (k + 3) * CM, CM)], wb[b], wsem[b])
        return carry

    lax.fori_loop(0, KGM, group, 0)

    for b in (0, 1):
        pltpu.make_async_copy(g_hbm.at[srcb[b]], rows[b], gsem[b]).wait()
        mul_chunk(b, wb[b])
        pltpu.make_async_copy(
            dst1.at[pl.ds(base, CM)], dstb[b], dstsem[b]).wait()
        pltpu.async_copy(rows[b], acc_sh.at[dstb[b]], ssem[b], add=True)
        b2 = (b + 2) % NBM
        pltpu.make_async_copy(rows[b2], acc_sh.at[dstb[b2]], ssem[b2]).wait()
    pltpu.make_async_copy(rows[1], acc_sh.at[dstb[1]], ssem[1]).wait()
    plsc.subcore_barrier()
    pltpu.sync_copy(acc_sh.at[pl.ds(zofs, ROWS_PER_SUB)],
                    acc_out.at[cid, pl.ds(zofs, ROWS_PER_SUB)])


def _tc_dis_body(deg_ref, out_ref):
    out_ref[...] = lax.rsqrt(1.0 + deg_ref[0:1, :] + deg_ref[1:2, :])


def _tc_dis(deg2):
    return pl.pallas_call(
        _tc_dis_body,
        out_shape=jax.ShapeDtypeStruct((1, N_PAD), jnp.float32),
    )(deg2)


_MM_BLK = 640


def _tc_mm_body(x_ref, w_ref, h_ref):
    h_ref[...] = jnp.dot(
        x_ref[...], w_ref[...], preferred_element_type=jnp.float32)


def _tc_mm(x_p, W):
    grid = (N_PAD // _MM_BLK,)
    return pl.pallas_call(
        _tc_mm_body,
        grid=grid,
        in_specs=[
            pl.BlockSpec((_MM_BLK, D), lambda i: (i, 0)),
            pl.BlockSpec((D, D), lambda i: (0, 0)),
        ],
        out_specs=pl.BlockSpec((_MM_BLK, D), lambda i: (i, 0)),
        out_shape=jax.ShapeDtypeStruct((N_PAD, D), jnp.float32),
    )(x_p, W)


_FIN_BLK = 400


def _tc_fin_body(acc_ref, h_ref, dis_ref, b_ref, pa_ref, out_ref):
    dis = dis_ref[...]
    o = dis * (acc_ref[0] + acc_ref[1] + dis * h_ref[...]) + b_ref[...]
    out_ref[...] = jnp.where(o >= 0.0, o, pa_ref[...] * o)


def _tc_fin(acc2, g, dis_col, b2, pa2):
    grid = (N // _FIN_BLK,)
    return pl.pallas_call(
        _tc_fin_body,
        grid=grid,
        in_specs=[
            pl.BlockSpec((NC, _FIN_BLK, D), lambda i: (0, i, 0)),
            pl.BlockSpec((_FIN_BLK, D), lambda i: (i, 0)),
            pl.BlockSpec((_FIN_BLK, 1), lambda i: (i, 0)),
            pl.BlockSpec((1, D), lambda i: (0, 0)),
            pl.BlockSpec((1, D), lambda i: (0, 0)),
        ],
        out_specs=pl.BlockSpec((_FIN_BLK, D), lambda i: (i, 0)),
        out_shape=jax.ShapeDtypeStruct((N, D), jnp.float32),
    )(acc2, g, dis_col, b2, pa2)


@jax.jit
def kernel(x, edge_index, edge_weight, W, b, prelu_a):
    src1 = edge_index[0]
    dst1 = edge_index[1]
    x_p = jnp.concatenate(
        [x, jnp.zeros((N_PAD - N, D), jnp.float32)], axis=0)

    h = _tc_mm(x_p, W)
    deg2 = _sc_deg(dst1, edge_weight)
    dis_row = _tc_dis(deg2)
    acc2 = _sc_msg(h, src1, dst1, edge_weight, dis_row.reshape(N_PAD))
    return _tc_fin(acc2, h[:N], dis_row.reshape(N_PAD, 1)[:N],
                   b.reshape(1, D), prelu_a.reshape(1, D))


# speedup vs baseline: 35.7389x; 1.0019x over previous
import functools

import jax
import jax.numpy as jnp
from jax import lax
from jax.experimental import pallas as pl
from jax.experimental.pallas import tpu as pltpu
from jax.experimental.pallas import tpu_sc as plsc

N = 10000
E = 320000
D = 128

NC = 2
NS = 16
NW = NC * NS

N_PAD = 10240
ROWS_PER_SUB = N_PAD // NS

CM = 80
NBM = 3
KM = E // (NW * CM)
KGM = (KM - 2) // NBM
EPW = E // NW

_mesh = plsc.VectorSubcoreMesh(core_axis_name="c", subcore_axis_name="s")
_sc_params = pltpu.CompilerParams(needs_layout_passes=False)


NBD = 4
KD2 = E // (NW * CM)


@functools.partial(
    pl.kernel,
    out_type=jax.ShapeDtypeStruct((NC, N_PAD), jnp.float32),
    mesh=_mesh,
    scratch_types=[
        [pltpu.VMEM((CM,), jnp.int32) for _ in range(NBD)],
        [pltpu.VMEM((CM,), jnp.float32) for _ in range(NBD)],
        [pltpu.SemaphoreType.DMA for _ in range(NBD)],
        [pltpu.SemaphoreType.DMA for _ in range(NBD)],
        [pltpu.SemaphoreType.DMA for _ in range(NBD)],
        pltpu.VMEM((ROWS_PER_SUB,), jnp.float32),
        pltpu.VMEM_SHARED((N_PAD,), jnp.float32),
    ],
    compiler_params=_sc_params,
)
def _sc_deg(dst1, w1, deg_out, dstb, wb, dsem, wsem, ssem, zline_v, deg_sh):
    cid = lax.axis_index("c")
    sid = lax.axis_index("s")
    wid = sid * NC + cid
    zofs = sid * ROWS_PER_SUB
    base = wid * EPW

    def zrow(i, carry):
        zline_v[pl.ds(i * 16, 16)] = jnp.zeros((16,), jnp.float32)
        return carry
    lax.fori_loop(0, ROWS_PER_SUB // 16, zrow, 0)
    pltpu.sync_copy(zline_v, deg_sh.at[pl.ds(zofs, ROWS_PER_SUB)])
    for b in range(2):
        pltpu.sync_copy(dst1.at[pl.ds(base + b * CM, CM)], dstb[b])
        pltpu.sync_copy(w1.at[pl.ds(base + b * CM, CM)], wb[b])
    plsc.subcore_barrier()

    def group(g, carry):
        for b in range(NBD):
            j = g * NBD + b
            b3 = (b + 2) % NBD
            @pl.when(j >= 2)
            def _():
                pltpu.make_async_copy(
                    dst1.at[pl.ds(base, CM)], dstb[b], dsem[b]).wait()
                pltpu.make_async_copy(
                    w1.at[pl.ds(base, CM)], wb[b], wsem[b]).wait()
            pltpu.async_copy(wb[b], deg_sh.at[dstb[b]], ssem[b], add=True)

            @pl.when(j >= 2)
            def _():
                pltpu.make_async_copy(
                    wb[b3], deg_sh.at[dstb[b3]], ssem[b3]).wait()

            @pl.when(j + 2 < KD2)
            def _():
                pltpu.async_copy(
                    dst1.at[pl.ds(base + (j + 2) * CM, CM)], dstb[b3],
                    dsem[b3])
                pltpu.async_copy(
                    w1.at[pl.ds(base + (j + 2) * CM, CM)], wb[b3], wsem[b3])
        return carry

    lax.fori_loop(0, (KD2 - 1) // NBD, group, 0)
    pltpu.make_async_copy(dst1.at[pl.ds(base, CM)], dstb[0], dsem[0]).wait()
    pltpu.make_async_copy(w1.at[pl.ds(base, CM)], wb[0], wsem[0]).wait()
    pltpu.async_copy(wb[0], deg_sh.at[dstb[0]], ssem[0], add=True)
    for b in (2, 3, 0):
        pltpu.make_async_copy(wb[b], deg_sh.at[dstb[b]], ssem[b]).wait()
    plsc.subcore_barrier()
    pltpu.sync_copy(deg_sh.at[pl.ds(zofs, ROWS_PER_SUB)],
                    deg_out.at[cid, pl.ds(zofs, ROWS_PER_SUB)])


@functools.partial(
    pl.kernel,
    out_type=jax.ShapeDtypeStruct((NC, N_PAD, D), jnp.float32),
    mesh=_mesh,
    scratch_types=[
        [pltpu.VMEM((CM, D), jnp.float32) for _ in range(NBM)],
        [pltpu.VMEM((CM,), jnp.int32) for _ in range(NBM)],
        [pltpu.VMEM((CM,), jnp.int32) for _ in range(NBM)],
        [pltpu.VMEM((CM,), jnp.float32) for _ in range(NBM)],
        [pltpu.SemaphoreType.DMA for _ in range(NBM)],
        [pltpu.SemaphoreType.DMA for _ in range(NBM)],
        [pltpu.SemaphoreType.DMA for _ in range(NBM)],
        [pltpu.SemaphoreType.DMA for _ in range(NBM)],
        [pltpu.SemaphoreType.DMA for _ in range(NBM)],
        pltpu.VMEM((N_PAD,), jnp.float32),
        pltpu.VMEM_SHARED((N_PAD, D), jnp.float32),
    ],
    compiler_params=_sc_params,
)
def _sc_msg(g_hbm, src1, dst1, w1, dis1, acc_out,
            rows, srcb, dstb, wb, gsem, ssem, srcsem, dstsem, wsem,
            dis_v, acc_sh):
    cid = lax.axis_index("c")
    sid = lax.axis_index("s")
    zofs = sid * ROWS_PER_SUB
    wid = sid * NC + cid
    base = wid * EPW

    def zrow(i, carry):
        for j in range(D // 16):
            rows[0][i, pl.ds(j * 16, 16)] = jnp.zeros((16,), jnp.float32)
        return carry
    lax.fori_loop(0, CM, zrow, 0)
    for i in range(ROWS_PER_SUB // CM):
        pltpu.sync_copy(rows[0], acc_sh.at[pl.ds(zofs + i * CM, CM)])

    pltpu.sync_copy(dis1, dis_v)
    for b in range(NBM):
        pltpu.sync_copy(src1.at[pl.ds(base + b * CM, CM)], srcb[b])
        pltpu.sync_copy(w1.at[pl.ds(base + b * CM, CM)], wb[b])
    for b in range(NBM - 1):
        pltpu.sync_copy(dst1.at[pl.ds(base + b * CM, CM)], dstb[b])
    plsc.subcore_barrier()
    for b in range(NBM - 1):
        pltpu.async_copy(g_hbm.at[srcb[b]], rows[b], gsem[b])

    def mul_chunk(b, ww):
        for i in range(CM // 16):
            sl = pl.ds(i * 16, 16)
            d16 = plsc.load_gather(dis_v, [srcb[b][sl]])
            ww[sl] = ww[sl] * d16

        @plsc.parallel_loop(0, CM, 1, unroll=4)
        def _(e):
            splat = plsc.load_gather(
                ww, [jnp.full((16,), 0, jnp.int32) + e])
            for j in range(D // 16):
                sl = pl.ds(j * 16, 16)
                rows[b][e, sl] = rows[b][e, sl] * splat

    def group(g, carry):
        for b in range(NBM):
            k = g * NBM + b
            b2 = (b + 2) % NBM
            pltpu.make_async_copy(g_hbm.at[srcb[b]], rows[b], gsem[b]).wait()
            mul_chunk(b, wb[b])

            @pl.when(k >= 2)
            def _():
                pltpu.make_async_copy(
                    dst1.at[pl.ds(base, CM)], dstb[b], dstsem[b]).wait()
            pltpu.async_copy(rows[b], acc_sh.at[dstb[b]], ssem[b], add=True)

            @pl.when(k >= 1)
            def _():
                pltpu.make_async_copy(
                    rows[b2], acc_sh.at[dstb[b2]], ssem[b2]).wait()

            @pl.when(k + 2 < KM)
            def _():
                @pl.when(k >= 1)
                def _():
                    pltpu.make_async_copy(
                        src1.at[pl.ds(base, CM)], srcb[b2], srcsem[b2]).wait()
                    pltpu.make_async_copy(
                        w1.at[pl.ds(base, CM)], wb[b2], wsem[b2]).wait()
                pltpu.async_copy(g_hbm.at[srcb[b2]], rows[b2], gsem[b2])
                pltpu.async_copy(
                    dst1.at[pl.ds(base + (k + 2) * CM, CM)],
                    dstb[b2], dstsem[b2])

            @pl.when(k + 3 < KM)
            def _():
                pltpu.async_copy(
                    src1.at[pl.ds(base + (k + 3) * CM, CM)], srcb[b], srcsem[b])
                pltpu.async_copy(
                    w1.at[pl.ds(base + (k + 3) * CM, CM)], wb[b], wsem[b])
        return carry

    lax.fori_loop(0, KGM, group, 0)

    for b in (0, 1):
        pltpu.make_async_copy(g_hbm.at[srcb[b]], rows[b], gsem[b]).wait()
        mul_chunk(b, wb[b])
        pltpu.make_async_copy(
            dst1.at[pl.ds(base, CM)], dstb[b], dstsem[b]).wait()
        pltpu.async_copy(rows[b], acc_sh.at[dstb[b]], ssem[b], add=True)
        b2 = (b + 2) % NBM
        pltpu.make_async_copy(rows[b2], acc_sh.at[dstb[b2]], ssem[b2]).wait()
    pltpu.make_async_copy(rows[1], acc_sh.at[dstb[1]], ssem[1]).wait()
    plsc.subcore_barrier()
    pltpu.sync_copy(acc_sh.at[pl.ds(zofs, ROWS_PER_SUB)],
                    acc_out.at[cid, pl.ds(zofs, ROWS_PER_SUB)])


def _tc_dis_body(deg_ref, out_ref):
    out_ref[...] = lax.rsqrt(1.0 + deg_ref[0:1, :] + deg_ref[1:2, :])


def _tc_dis(deg2):
    return pl.pallas_call(
        _tc_dis_body,
        out_shape=jax.ShapeDtypeStruct((1, N_PAD), jnp.float32),
    )(deg2)


_MM_BLK = 640


def _tc_mm_body(x_ref, w_ref, h_ref):
    h_ref[...] = jnp.dot(
        x_ref[...], w_ref[...], preferred_element_type=jnp.float32)


def _tc_mm(x_p, W):
    grid = (N_PAD // _MM_BLK,)
    return pl.pallas_call(
        _tc_mm_body,
        grid=grid,
        in_specs=[
            pl.BlockSpec((_MM_BLK, D), lambda i: (i, 0)),
            pl.BlockSpec((D, D), lambda i: (0, 0)),
        ],
        out_specs=pl.BlockSpec((_MM_BLK, D), lambda i: (i, 0)),
        out_shape=jax.ShapeDtypeStruct((N_PAD, D), jnp.float32),
    )(x_p, W)


_FIN_BLK = 400


def _tc_fin_body(acc_ref, h_ref, dis_ref, b_ref, pa_ref, out_ref):
    dis = dis_ref[...]
    o = dis * (acc_ref[0] + acc_ref[1] + dis * h_ref[...]) + b_ref[...]
    out_ref[...] = jnp.where(o >= 0.0, o, pa_ref[...] * o)


def _tc_fin(acc2, g, dis_col, b2, pa2):
    grid = (N // _FIN_BLK,)
    return pl.pallas_call(
        _tc_fin_body,
        grid=grid,
        in_specs=[
            pl.BlockSpec((NC, _FIN_BLK, D), lambda i: (0, i, 0)),
            pl.BlockSpec((_FIN_BLK, D), lambda i: (i, 0)),
            pl.BlockSpec((_FIN_BLK, 1), lambda i: (i, 0)),
            pl.BlockSpec((1, D), lambda i: (0, 0)),
            pl.BlockSpec((1, D), lambda i: (0, 0)),
        ],
        out_specs=pl.BlockSpec((_FIN_BLK, D), lambda i: (i, 0)),
        out_shape=jax.ShapeDtypeStruct((N, D), jnp.float32),
    )(acc2, g, dis_col, b2, pa2)


@jax.jit
def kernel(x, edge_index, edge_weight, W, b, prelu_a):
    src1 = edge_index[0]
    dst1 = edge_index[1]
    x_p = jnp.concatenate(
        [x, jnp.zeros((N_PAD - N, D), jnp.float32)], axis=0)

    h = _tc_mm(x_p, W)
    deg2 = _sc_deg(dst1, edge_weight)
    dis_row = _tc_dis(deg2)
    acc2 = _sc_msg(h, src1, dst1, edge_weight, dis_row.reshape(N_PAD))
    return _tc_fin(acc2, h[:N], dis_row.reshape(N_PAD, 1)[:N],
                   b.reshape(1, D), prelu_a.reshape(1, D))


# speedup vs baseline: 36.0912x; 1.0099x over previous
import functools

import jax
import jax.numpy as jnp
from jax import lax
from jax.experimental import pallas as pl
from jax.experimental.pallas import tpu as pltpu
from jax.experimental.pallas import tpu_sc as plsc

N = 10000
E = 320000
D = 128

NC = 2
NS = 16
NW = NC * NS

N_PAD = 10240
ROWS_PER_SUB = N_PAD // NS

CM = 80
NBM = 3
KM = E // (NW * CM)
KGM = (KM - 2) // NBM
EPW = E // NW

_mesh = plsc.VectorSubcoreMesh(core_axis_name="c", subcore_axis_name="s")
_sc_params = pltpu.CompilerParams(needs_layout_passes=False)


NBD = 4
KD2 = E // (NW * CM)


@functools.partial(
    pl.kernel,
    out_type=jax.ShapeDtypeStruct((NC, N_PAD), jnp.float32),
    mesh=_mesh,
    scratch_types=[
        [pltpu.VMEM((CM,), jnp.int32) for _ in range(NBD)],
        [pltpu.VMEM((CM,), jnp.float32) for _ in range(NBD)],
        [pltpu.SemaphoreType.DMA for _ in range(NBD)],
        [pltpu.SemaphoreType.DMA for _ in range(NBD)],
        [pltpu.SemaphoreType.DMA for _ in range(NBD)],
        pltpu.VMEM((ROWS_PER_SUB,), jnp.float32),
        pltpu.VMEM_SHARED((N_PAD,), jnp.float32),
    ],
    compiler_params=_sc_params,
)
def _sc_deg(dst1, w1, deg_out, dstb, wb, dsem, wsem, ssem, zline_v, deg_sh):
    cid = lax.axis_index("c")
    sid = lax.axis_index("s")
    wid = sid * NC + cid
    zofs = sid * ROWS_PER_SUB
    base = wid * EPW

    def zrow(i, carry):
        zline_v[pl.ds(i * 16, 16)] = jnp.zeros((16,), jnp.float32)
        return carry
    lax.fori_loop(0, ROWS_PER_SUB // 16, zrow, 0)
    pltpu.sync_copy(zline_v, deg_sh.at[pl.ds(zofs, ROWS_PER_SUB)])
    for b in range(2):
        pltpu.sync_copy(dst1.at[pl.ds(base + b * CM, CM)], dstb[b])
        pltpu.sync_copy(w1.at[pl.ds(base + b * CM, CM)], wb[b])
    plsc.subcore_barrier()

    def group(g, carry):
        for b in range(NBD):
            j = g * NBD + b
            b3 = (b + 2) % NBD
            @pl.when(j >= 2)
            def _():
                pltpu.make_async_copy(
                    dst1.at[pl.ds(base, CM)], dstb[b], dsem[b]).wait()
                pltpu.make_async_copy(
                    w1.at[pl.ds(base, CM)], wb[b], wsem[b]).wait()
            pltpu.async_copy(wb[b], deg_sh.at[dstb[b]], ssem[b], add=True)

            @pl.when(j >= 2)
            def _():
                pltpu.make_async_copy(
                    wb[b3], deg_sh.at[dstb[b3]], ssem[b3]).wait()

            @pl.when(j + 2 < KD2)
            def _():
                pltpu.async_copy(
                    dst1.at[pl.ds(base + (j + 2) * CM, CM)], dstb[b3],
                    dsem[b3])
                pltpu.async_copy(
                    w1.at[pl.ds(base + (j + 2) * CM, CM)], wb[b3], wsem[b3])
        return carry

    lax.fori_loop(0, (KD2 - 1) // NBD, group, 0)
    pltpu.make_async_copy(dst1.at[pl.ds(base, CM)], dstb[0], dsem[0]).wait()
    pltpu.make_async_copy(w1.at[pl.ds(base, CM)], wb[0], wsem[0]).wait()
    pltpu.async_copy(wb[0], deg_sh.at[dstb[0]], ssem[0], add=True)
    for b in (2, 3, 0):
        pltpu.make_async_copy(wb[b], deg_sh.at[dstb[b]], ssem[b]).wait()
    plsc.subcore_barrier()
    pltpu.sync_copy(deg_sh.at[pl.ds(zofs, ROWS_PER_SUB)],
                    deg_out.at[cid, pl.ds(zofs, ROWS_PER_SUB)])


@functools.partial(
    pl.kernel,
    out_type=jax.ShapeDtypeStruct((NC, N_PAD, D), jnp.float32),
    mesh=_mesh,
    scratch_types=[
        [pltpu.VMEM((CM, D), jnp.float32) for _ in range(NBM)],
        [pltpu.VMEM((CM,), jnp.int32) for _ in range(NBM)],
        [pltpu.VMEM((CM,), jnp.int32) for _ in range(NBM)],
        [pltpu.VMEM((CM,), jnp.float32) for _ in range(NBM)],
        [pltpu.SemaphoreType.DMA for _ in range(NBM)],
        [pltpu.SemaphoreType.DMA for _ in range(NBM)],
        [pltpu.SemaphoreType.DMA for _ in range(NBM)],
        [pltpu.SemaphoreType.DMA for _ in range(NBM)],
        [pltpu.SemaphoreType.DMA for _ in range(NBM)],
        pltpu.VMEM((N_PAD,), jnp.float32),
        pltpu.VMEM_SHARED((N_PAD, D), jnp.float32),
    ],
    compiler_params=_sc_params,
)
def _sc_msg(g_hbm, src1, dst1, w1, dis1, acc_out,
            rows, srcb, dstb, wb, gsem, ssem, srcsem, dstsem, wsem,
            dis_v, acc_sh):
    cid = lax.axis_index("c")
    sid = lax.axis_index("s")
    zofs = sid * ROWS_PER_SUB
    wid = sid * NC + cid
    base = wid * EPW

    def zrow(i, carry):
        for j in range(D // 16):
            rows[0][i, pl.ds(j * 16, 16)] = jnp.zeros((16,), jnp.float32)
        return carry
    lax.fori_loop(0, CM, zrow, 0)
    for i in range(ROWS_PER_SUB // CM):
        pltpu.sync_copy(rows[0], acc_sh.at[pl.ds(zofs + i * CM, CM)])

    pltpu.sync_copy(dis1, dis_v)
    for b in range(NBM):
        pltpu.sync_copy(src1.at[pl.ds(base + b * CM, CM)], srcb[b])
        pltpu.sync_copy(w1.at[pl.ds(base + b * CM, CM)], wb[b])
    for b in range(NBM - 1):
        pltpu.sync_copy(dst1.at[pl.ds(base + b * CM, CM)], dstb[b])
    plsc.subcore_barrier()
    for b in range(NBM - 1):
        pltpu.async_copy(g_hbm.at[srcb[b]], rows[b], gsem[b])

    def mul_chunk(b, ww):
        for i in range(CM // 16):
            sl = pl.ds(i * 16, 16)
            d16 = plsc.load_gather(dis_v, [srcb[b][sl]])
            ww[sl] = ww[sl] * d16

        @plsc.parallel_loop(0, CM, 1, unroll=4)
        def _(e):
            splat = plsc.load_gather(
                ww, [jnp.full((16,), 0, jnp.int32) + e])
            for j in range(D // 16):
                sl = pl.ds(j * 16, 16)
                rows[b][e, sl] = rows[b][e, sl] * splat

    def group(g, carry):
        for b in range(NBM):
            k = g * NBM + b
            b2 = (b + 2) % NBM
            pltpu.make_async_copy(g_hbm.at[srcb[b]], rows[b], gsem[b]).wait()
            mul_chunk(b, wb[b])

            @pl.when(k >= 2)
            def _():
                pltpu.make_async_copy(
                    dst1.at[pl.ds(base, CM)], dstb[b], dstsem[b]).wait()
            pltpu.async_copy(rows[b], acc_sh.at[dstb[b]], ssem[b], add=True)

            @pl.when(k >= 1)
            def _():
                pltpu.make_async_copy(
                    rows[b2], acc_sh.at[dstb[b2]], ssem[b2]).wait()

            @pl.when(k + 2 < KM)
            def _():
                @pl.when(k >= 1)
                def _():
                    pltpu.make_async_copy(
                        src1.at[pl.ds(base, CM)], srcb[b2], srcsem[b2]).wait()
                    pltpu.make_async_copy(
                        w1.at[pl.ds(base, CM)], wb[b2], wsem[b2]).wait()
                pltpu.async_copy(g_hbm.at[srcb[b2]], rows[b2], gsem[b2])
                pltpu.async_copy(
                    dst1.at[pl.ds(base + (k + 2) * CM, CM)],
                    dstb[b2], dstsem[b2])

            @pl.when(k + 3 < KM)
            def _():
                pltpu.async_copy(
                    src1.at[pl.ds(base + (k + 3) * CM, CM)], srcb[b], srcsem[b])
                pltpu.async_copy(
                    w1.at[pl.ds(base + (k + 3) * CM, CM)], wb[b], wsem[b])
        return carry

    lax.fori_loop(0, KGM, group, 0)

    for b in (0, 1):
        pltpu.make_async_copy(g_hbm.at[srcb[b]], rows[b], gsem[b]).wait()
        mul_chunk(b, wb[b])
        pltpu.make_async_copy(
            dst1.at[pl.ds(base, CM)], dstb[b], dstsem[b]).wait()
        pltpu.async_copy(rows[b], acc_sh.at[dstb[b]], ssem[b], add=True)
        b2 = (b + 2) % NBM
        pltpu.make_async_copy(rows[b2], acc_sh.at[dstb[b2]], ssem[b2]).wait()
    pltpu.make_async_copy(rows[1], acc_sh.at[dstb[1]], ssem[1]).wait()
    plsc.subcore_barrier()
    pltpu.sync_copy(acc_sh.at[pl.ds(zofs, ROWS_PER_SUB)],
                    acc_out.at[cid, pl.ds(zofs, ROWS_PER_SUB)])


def _tc_dis_body(deg_ref, out_ref):
    out_ref[...] = lax.rsqrt(1.0 + deg_ref[0:1, :] + deg_ref[1:2, :])


def _tc_dis(deg2):
    return pl.pallas_call(
        _tc_dis_body,
        out_shape=jax.ShapeDtypeStruct((1, N_PAD), jnp.float32),
    )(deg2)


_MM_BLK = 400


def _tc_mm_body(x_ref, w_ref, h_ref):
    h_ref[...] = jnp.dot(
        x_ref[...], w_ref[...], preferred_element_type=jnp.float32)


def _tc_mm(x, W):
    grid = (N // _MM_BLK,)
    return pl.pallas_call(
        _tc_mm_body,
        grid=grid,
        in_specs=[
            pl.BlockSpec((_MM_BLK, D), lambda i: (i, 0)),
            pl.BlockSpec((D, D), lambda i: (0, 0)),
        ],
        out_specs=pl.BlockSpec((_MM_BLK, D), lambda i: (i, 0)),
        out_shape=jax.ShapeDtypeStruct((N, D), jnp.float32),
    )(x, W)


_FIN_BLK = 400


def _tc_fin_body(acc_ref, h_ref, dis_ref, b_ref, pa_ref, out_ref):
    dis = dis_ref[...]
    o = dis * (acc_ref[0] + acc_ref[1] + dis * h_ref[...]) + b_ref[...]
    out_ref[...] = jnp.where(o >= 0.0, o, pa_ref[...] * o)


def _tc_fin(acc2, g, dis_col, b2, pa2):
    grid = (N // _FIN_BLK,)
    return pl.pallas_call(
        _tc_fin_body,
        grid=grid,
        in_specs=[
            pl.BlockSpec((NC, _FIN_BLK, D), lambda i: (0, i, 0)),
            pl.BlockSpec((_FIN_BLK, D), lambda i: (i, 0)),
            pl.BlockSpec((_FIN_BLK, 1), lambda i: (i, 0)),
            pl.BlockSpec((1, D), lambda i: (0, 0)),
            pl.BlockSpec((1, D), lambda i: (0, 0)),
        ],
        out_specs=pl.BlockSpec((_FIN_BLK, D), lambda i: (i, 0)),
        out_shape=jax.ShapeDtypeStruct((N, D), jnp.float32),
    )(acc2, g, dis_col, b2, pa2)


@jax.jit
def kernel(x, edge_index, edge_weight, W, b, prelu_a):
    src1 = edge_index[0]
    dst1 = edge_index[1]
    h = _tc_mm(x, W)
    deg2 = _sc_deg(dst1, edge_weight)
    dis_row = _tc_dis(deg2)
    acc2 = _sc_msg(h, src1, dst1, edge_weight, dis_row.reshape(N_PAD))
    return _tc_fin(acc2, h, dis_row.reshape(N_PAD, 1)[:N],
                   b.reshape(1, D), prelu_a.reshape(1, D))
